# baseline (speedup 1.0000x reference)
import functools

import jax
import jax.numpy as jnp
from jax import lax
from jax.experimental import pallas as pl
from jax.experimental.pallas import tpu as pltpu
from jax.experimental.pallas import tpu_sc as plsc

_ROWS = 4096
_COLS = 200
_D = 64
_B = _ROWS * _COLS
_NC = 2
_NS = 16
_NW = _NC * _NS
_BPW = _B // _NW
_C = 640
_NCHUNK = _BPW // _C


def _gather_body(x_hbm, W_hbm, out_hbm, idx_v, buf0, buf1, sg0, sg1, sw0, sw1):
    wid = lax.axis_index("s") * _NC + lax.axis_index("c")
    base = wid * _BPW
    bufs = (buf0, buf1)
    sgs = (sg0, sg1)
    sws = (sw0, sw1)

    pltpu.sync_copy(x_hbm.at[pl.ds(base, _BPW)], idx_v)

    def start_gather(b, k):
        pltpu.async_copy(W_hbm.at[idx_v.at[pl.ds(k * _C, _C)]], bufs[b], sgs[b])

    def start_wb(b, k):
        pltpu.async_copy(bufs[b], out_hbm.at[pl.ds(base + k * _C, _C)], sws[b])

    def wait_gather(b):
        pltpu.make_async_copy(W_hbm.at[pl.ds(0, _C)], bufs[b], sgs[b]).wait()

    def wait_wb(b):
        pltpu.make_async_copy(bufs[b], out_hbm.at[pl.ds(base, _C)], sws[b]).wait()

    for b in range(2):
        start_gather(b, b)

    def step(i, carry):
        for b in range(2):
            k = 2 * i + b
            wait_gather(b)
            start_wb(b, k)
        for b in range(2):
            wait_wb(b)
            start_gather(b, 2 * i + 2 + b)
        return carry

    lax.fori_loop(0, _NCHUNK // 2 - 1, step, 0)

    for b in range(2):
        k = _NCHUNK - 2 + b
        wait_gather(b)
        start_wb(b, k)
    for b in range(2):
        wait_wb(b)


_gather = functools.partial(
    pl.kernel,
    out_type=jax.ShapeDtypeStruct((_B, _D), jnp.float32),
    mesh=plsc.VectorSubcoreMesh(core_axis_name="c", subcore_axis_name="s"),
    scratch_types=[
        pltpu.VMEM((_BPW,), jnp.int32),
        pltpu.VMEM((_C, _D), jnp.float32),
        pltpu.VMEM((_C, _D), jnp.float32),
        pltpu.SemaphoreType.DMA,
        pltpu.SemaphoreType.DMA,
        pltpu.SemaphoreType.DMA,
        pltpu.SemaphoreType.DMA,
    ],
    compiler_params=pltpu.CompilerParams(use_tc_tiling_on_sc=False),
)(_gather_body)


def _mask_body(x_ref, m_ref):
    m_ref[...] = x_ref[...] != 0


_mask = pl.pallas_call(
    _mask_body,
    out_shape=jax.ShapeDtypeStruct((_ROWS, _COLS), jnp.bool_),
)


def kernel(x, W):
    out = _gather(x.reshape(_B), W)
    pad_mask = _mask(x)
    return out.reshape(_ROWS, _COLS, _D), pad_mask


# speedup vs baseline: 1.3280x; 1.3280x over previous
import functools

import jax
import jax.numpy as jnp
from jax import lax
from jax.experimental import pallas as pl
from jax.experimental.pallas import tpu as pltpu
from jax.experimental.pallas import tpu_sc as plsc

_ROWS = 4096
_COLS = 200
_D = 64
_B = _ROWS * _COLS
_NC = 2
_NS = 16
_NW = _NC * _NS
_BPW = _B // _NW
_C = 640
_NCHUNK = _BPW // _C


def _gather_body(x_hbm, W_hbm, out_hbm, idx_v, buf0, buf1, sg0, sg1, sw0, sw1):
    wid = lax.axis_index("s") * _NC + lax.axis_index("c")
    base = wid * _BPW
    bufs = (buf0, buf1)
    sgs = (sg0, sg1)
    sws = (sw0, sw1)

    pltpu.sync_copy(x_hbm.at[pl.ds(base, _BPW)], idx_v)

    def start_gather(b, k):
        pltpu.async_copy(W_hbm.at[idx_v.at[pl.ds(k * _C, _C)]], bufs[b], sgs[b])

    def start_wb(b, k):
        pltpu.async_copy(
            bufs[b], out_hbm.at[pl.ds(base + k * _C, _C), pl.ds(0, _D)], sws[b]
        )

    def wait_gather(b):
        pltpu.make_async_copy(W_hbm.at[pl.ds(0, _C)], bufs[b], sgs[b]).wait()

    def wait_wb(b):
        pltpu.make_async_copy(
            bufs[b], out_hbm.at[pl.ds(base, _C), pl.ds(0, _D)], sws[b]
        ).wait()

    for b in range(2):
        start_gather(b, b)

    def step(i, carry):
        for b in range(2):
            k = 2 * i + b
            wait_gather(b)
            start_wb(b, k)
        for b in range(2):
            wait_wb(b)
            start_gather(b, 2 * i + 2 + b)
        return carry

    lax.fori_loop(0, _NCHUNK // 2 - 1, step, 0)

    for b in range(2):
        k = _NCHUNK - 2 + b
        wait_gather(b)
        start_wb(b, k)
    for b in range(2):
        wait_wb(b)


_gather = functools.partial(
    pl.kernel,
    out_type=jax.ShapeDtypeStruct((_B, 128), jnp.float32),
    mesh=plsc.VectorSubcoreMesh(core_axis_name="c", subcore_axis_name="s"),
    scratch_types=[
        pltpu.VMEM((_BPW,), jnp.int32),
        pltpu.VMEM((_C, _D), jnp.float32),
        pltpu.VMEM((_C, _D), jnp.float32),
        pltpu.SemaphoreType.DMA,
        pltpu.SemaphoreType.DMA,
        pltpu.SemaphoreType.DMA,
        pltpu.SemaphoreType.DMA,
    ],
    compiler_params=pltpu.CompilerParams(use_tc_tiling_on_sc=False),
)(_gather_body)


def _mask_body(x_ref, m_ref):
    m_ref[...] = x_ref[...] != 0


_mask = pl.pallas_call(
    _mask_body,
    out_shape=jax.ShapeDtypeStruct((_ROWS, _COLS), jnp.bool_),
)


def kernel(x, W):
    out = _gather(x.reshape(_B), W)
    pad_mask = _mask(x)
    return out.reshape(_ROWS, _COLS, 128)[..., :_D], pad_mask


# speedup vs baseline: 1.3286x; 1.0005x over previous
import functools

import jax
import jax.numpy as jnp
from jax import lax
from jax.experimental import pallas as pl
from jax.experimental.pallas import tpu as pltpu
from jax.experimental.pallas import tpu_sc as plsc

_VOC = 1000000
_ROWS = 4096
_COLS = 200
_D = 64
_B = _ROWS * _COLS
_NC = 2
_NS = 16
_NW = _NC * _NS
_BPW = _B // _NW
_C = 800
_NCHUNK = _BPW // _C


def _gather_body(x_hbm, W_hbm, out_hbm, idx_v, buf0, buf1, sg0, sg1, sw0, sw1):
    wid = lax.axis_index("s") * _NC + lax.axis_index("c")
    base = wid * _BPW
    bufs = (buf0, buf1)
    sgs = (sg0, sg1)
    sws = (sw0, sw1)

    pltpu.sync_copy(x_hbm.at[pl.ds(base, _BPW)], idx_v)

    def start_gather(b, k):
        pltpu.async_copy(W_hbm.at[idx_v.at[pl.ds(k * _C, _C)]], bufs[b], sgs[b])

    def start_wb(b, k):
        pltpu.async_copy(
            bufs[b], out_hbm.at[pl.ds(base + k * _C, _C), pl.ds(0, _D)], sws[b]
        )

    def wait_gather(b):
        pltpu.make_async_copy(W_hbm.at[pl.ds(0, _C)], bufs[b], sgs[b]).wait()

    def wait_wb(b):
        pltpu.make_async_copy(
            bufs[b], out_hbm.at[pl.ds(base, _C), pl.ds(0, _D)], sws[b]
        ).wait()

    for b in range(2):
        start_gather(b, b)

    def step(i, carry):
        for b in range(2):
            k = 2 * i + b
            wait_gather(b)
            start_wb(b, k)
        for b in range(2):
            wait_wb(b)
            start_gather(b, 2 * i + 2 + b)
        return carry

    lax.fori_loop(0, _NCHUNK // 2 - 1, step, 0)

    for b in range(2):
        k = _NCHUNK - 2 + b
        wait_gather(b)
        start_wb(b, k)
    for b in range(2):
        wait_wb(b)


_gather = functools.partial(
    pl.kernel,
    out_type=jax.ShapeDtypeStruct((_B, 128), jnp.float32),
    mesh=plsc.VectorSubcoreMesh(core_axis_name="c", subcore_axis_name="s"),
    scratch_types=[
        pltpu.VMEM((_BPW,), jnp.int32),
        pltpu.VMEM((_C, _D), jnp.float32),
        pltpu.VMEM((_C, _D), jnp.float32),
        pltpu.SemaphoreType.DMA,
        pltpu.SemaphoreType.DMA,
        pltpu.SemaphoreType.DMA,
        pltpu.SemaphoreType.DMA,
    ],
    compiler_params=pltpu.CompilerParams(use_tc_tiling_on_sc=False),
)(_gather_body)


def _mask_body(x_ref, m_ref):
    m_ref[...] = x_ref[...] != 0


_mask = pl.pallas_call(
    _mask_body,
    out_shape=jax.ShapeDtypeStruct((_ROWS, _COLS), jnp.bool_),
)


def kernel(x, W):
    out = _gather(x.reshape(_B), W)
    pad_mask = _mask(x)
    return out.reshape(_ROWS, _COLS, 128)[..., :_D], pad_mask
